# baseline (speedup 1.0000x reference)
import functools
import math

import jax
import jax.numpy as jnp
from jax import lax
from jax.experimental import pallas as pl
from jax.experimental.pallas import tpu as pltpu
from jax.experimental.pallas import tpu_sc as plsc

_VOCAB = 1000000
_EMBED = 32
_B = 16384
_L = 50
_NC = 2
_NS = 16
_NW = _NC * _NS
_S = 2
_BC = _B // _S
_R = 16
_NBUF = 3
_AHEAD = _NBUF - 1
_SCALE = math.sqrt(_EMBED)

_mesh = plsc.VectorSubcoreMesh(core_axis_name="c", subcore_axis_name="s")


def _make_chunk(nrows):
    rw = nrows // _NW
    nblk = rw // _R

    @functools.partial(
        pl.kernel,
        mesh=_mesh,
        out_type=jax.ShapeDtypeStruct((nrows, _L, _EMBED), jnp.float32),
        scratch_types=[
            pltpu.VMEM((_NBUF, _R, _L), jnp.int32),
            pltpu.VMEM((_NBUF, _R, _L, _EMBED), jnp.float32),
            pltpu.SemaphoreType.DMA((_NBUF,)),
            pltpu.SemaphoreType.DMA((_NBUF,)),
        ],
        compiler_params=pltpu.CompilerParams(use_tc_tiling_on_sc=False),
    )
    def _embed_gather(x_hbm, table_hbm, out_hbm, idx_v, rows_v, gsem, osem):
        wid = lax.axis_index("s") * _NC + lax.axis_index("c")
        row0 = wid * rw

        def issue_block(j, slot):
            r0 = row0 + j * _R
            pltpu.sync_copy(x_hbm.at[pl.ds(r0, _R)], idx_v.at[slot])
            for r in range(_R):
                pltpu.async_copy(
                    table_hbm.at[idx_v.at[slot, r]],
                    rows_v.at[slot, r],
                    gsem.at[slot],
                )

        for j in range(_AHEAD):
            issue_block(j, j)

        def step(g, carry):
            slot = lax.rem(g, _NBUF)
            pltpu.make_async_copy(
                out_hbm.at[pl.ds(0, _R)], rows_v.at[slot], gsem.at[slot]
            ).wait()

            def scale_row(r, c2):
                @plsc.parallel_loop(0, _L, 1, unroll=5)
                def _scale(i):
                    rows_v[slot, r, i, pl.ds(0, 16)] = (
                        rows_v[slot, r, i, pl.ds(0, 16)] * _SCALE)
                    rows_v[slot, r, i, pl.ds(16, 16)] = (
                        rows_v[slot, r, i, pl.ds(16, 16)] * _SCALE)

                return c2

            lax.fori_loop(0, _R, scale_row, 0)

            r0 = row0 + g * _R
            pltpu.async_copy(rows_v.at[slot], out_hbm.at[pl.ds(r0, _R)],
                             osem.at[slot])

            j = g + _AHEAD

            @pl.when(j < nblk)
            def _prefetch():
                s2 = lax.rem(j, _NBUF)

                @pl.when(g >= 1)
                def _wait_prev_scatter():
                    pltpu.make_async_copy(
                        rows_v.at[s2], out_hbm.at[pl.ds(0, _R)], osem.at[s2]
                    ).wait()

                issue_block(j, s2)

            return carry

        lax.fori_loop(0, nblk, step, 0)

        for s in range(_NBUF):
            pltpu.make_async_copy(
                rows_v.at[s], out_hbm.at[pl.ds(0, _R)], osem.at[s]
            ).wait()

    return _embed_gather


_chunk_kernel = _make_chunk(_BC)


def kernel(x, table):
    xi = x.astype(jnp.int32)
    outs = [_chunk_kernel(xi[c * _BC:(c + 1) * _BC], table)
            for c in range(_S)]
    return jnp.concatenate(outs, axis=0)


# speedup vs baseline: 1.0337x; 1.0337x over previous
import functools
import math

import jax
import jax.numpy as jnp
from jax import lax
from jax.experimental import pallas as pl
from jax.experimental.pallas import tpu as pltpu
from jax.experimental.pallas import tpu_sc as plsc

_VOCAB = 1000000
_EMBED = 32
_B = 16384
_L = 50
_NC = 2
_NS = 16
_NW = _NC * _NS
_S = 4
_BC = _B // _S
_R = 16
_NBUF = 3
_AHEAD = _NBUF - 1
_SCALE = math.sqrt(_EMBED)

_mesh = plsc.VectorSubcoreMesh(core_axis_name="c", subcore_axis_name="s")


def _make_chunk(nrows):
    rw = nrows // _NW
    nblk = rw // _R

    @functools.partial(
        pl.kernel,
        mesh=_mesh,
        out_type=jax.ShapeDtypeStruct((nrows, _L, _EMBED), jnp.float32),
        scratch_types=[
            pltpu.VMEM((_NBUF, _R, _L), jnp.int32),
            pltpu.VMEM((_NBUF, _R, _L, _EMBED), jnp.float32),
            pltpu.SemaphoreType.DMA((_NBUF,)),
            pltpu.SemaphoreType.DMA((_NBUF,)),
        ],
        compiler_params=pltpu.CompilerParams(use_tc_tiling_on_sc=False),
    )
    def _embed_gather(x_hbm, table_hbm, out_hbm, idx_v, rows_v, gsem, osem):
        wid = lax.axis_index("s") * _NC + lax.axis_index("c")
        row0 = wid * rw

        def issue_block(j, slot):
            r0 = row0 + j * _R
            pltpu.sync_copy(x_hbm.at[pl.ds(r0, _R)], idx_v.at[slot])
            for r in range(_R):
                pltpu.async_copy(
                    table_hbm.at[idx_v.at[slot, r]],
                    rows_v.at[slot, r],
                    gsem.at[slot],
                )

        for j in range(_AHEAD):
            issue_block(j, j)

        def step(g, carry):
            slot = lax.rem(g, _NBUF)
            pltpu.make_async_copy(
                out_hbm.at[pl.ds(0, _R)], rows_v.at[slot], gsem.at[slot]
            ).wait()

            def scale_row(r, c2):
                @plsc.parallel_loop(0, _L, 1, unroll=5)
                def _scale(i):
                    rows_v[slot, r, i, pl.ds(0, 16)] = (
                        rows_v[slot, r, i, pl.ds(0, 16)] * _SCALE)
                    rows_v[slot, r, i, pl.ds(16, 16)] = (
                        rows_v[slot, r, i, pl.ds(16, 16)] * _SCALE)

                return c2

            lax.fori_loop(0, _R, scale_row, 0)

            r0 = row0 + g * _R
            pltpu.async_copy(rows_v.at[slot], out_hbm.at[pl.ds(r0, _R)],
                             osem.at[slot])

            j = g + _AHEAD

            @pl.when(j < nblk)
            def _prefetch():
                s2 = lax.rem(j, _NBUF)

                @pl.when(g >= 1)
                def _wait_prev_scatter():
                    pltpu.make_async_copy(
                        rows_v.at[s2], out_hbm.at[pl.ds(0, _R)], osem.at[s2]
                    ).wait()

                issue_block(j, s2)

            return carry

        lax.fori_loop(0, nblk, step, 0)

        for s in range(_NBUF):
            pltpu.make_async_copy(
                rows_v.at[s], out_hbm.at[pl.ds(0, _R)], osem.at[s]
            ).wait()

    return _embed_gather


_chunk_kernel = _make_chunk(_BC)


def kernel(x, table):
    xi = x.astype(jnp.int32)
    outs = [_chunk_kernel(xi[c * _BC:(c + 1) * _BC], table)
            for c in range(_S)]
    return jnp.concatenate(outs, axis=0)
